# baseline (speedup 1.0000x reference)
import functools
import math

import jax
import jax.numpy as jnp
from jax import lax
from jax.experimental import pallas as pl
from jax.experimental.pallas import tpu as pltpu
from jax.experimental.pallas import tpu_sc as plsc

D_MODEL = 64
_SCALE = math.sqrt(D_MODEL)
_LANES = 128


@functools.lru_cache(maxsize=None)
def _build(V, D, B):
    info = plsc.get_sparse_core_info()
    NC, NS, L = info.num_cores, info.num_subcores, info.num_lanes
    NW = NC * NS
    assert B % NW == 0
    b_per_w = B // NW
    C = 128
    assert b_per_w % C == 0
    n_chunks = b_per_w // C
    NG = 4
    NWB = 2
    LOOK = 2
    assert n_chunks % NG == 0
    mesh = plsc.VectorSubcoreMesh(core_axis_name="c", subcore_axis_name="s")

    @functools.partial(
        pl.kernel,
        mesh=mesh,
        out_type=jax.ShapeDtypeStruct((B * D,), jnp.float32),
        compiler_params=pltpu.CompilerParams(use_tc_tiling_on_sc=True),
        scratch_types=[
            pltpu.VMEM((n_chunks, C), jnp.int32),
            pltpu.VMEM((NG, C, _LANES), jnp.float32),
            pltpu.VMEM((NWB, C * D), jnp.float32),
            [pltpu.SemaphoreType.DMA] * NG,
            [pltpu.SemaphoreType.DMA] * NWB,
        ],
    )
    def emb_kernel(table_hbm, x_hbm, out_hbm, idx_v, gbuf, wbuf, gsems, wsems):
        wid = lax.axis_index("s") * NC + lax.axis_index("c")
        base = wid * b_per_w
        pltpu.sync_copy(x_hbm.at[wid], idx_v)

        def start_gather(ci, b):
            pltpu.async_copy(table_hbm.at[idx_v.at[ci]], gbuf.at[b], gsems[b])

        def wait_gather(ci, b):
            pltpu.make_async_copy(
                table_hbm.at[idx_v.at[ci]], gbuf.at[b], gsems[b]
            ).wait()

        def wait_write(b):
            pltpu.make_async_copy(
                wbuf.at[b], out_hbm.at[pl.ds(base * D, C * D)], wsems[b]
            ).wait()

        def start_write(ci, b):
            pltpu.async_copy(
                wbuf.at[b],
                out_hbm.at[pl.ds((base + ci * C) * D, C * D)],
                wsems[b],
            )

        def scale(gb, wb):
            @plsc.parallel_loop(0, C, unroll=8)
            def _scale_body(r):
                for d in range(D // L):
                    wbuf[wb, pl.ds(r * D + d * L, L)] = (
                        gbuf[gb, r, pl.ds(d * L, L)] * _SCALE
                    )

        for ci in range(LOOK):
            start_gather(ci, ci % NG)

        for ci in range(NWB):
            start_gather(ci + LOOK, (ci + LOOK) % NG)
            wait_gather(ci, ci % NG)
            scale(ci % NG, ci % NWB)
            start_write(ci, ci % NWB)

        def steady(ci0, carry):
            for k in range(NG):
                ci = ci0 + k
                gb = (NWB + k) % NG
                gb_next = (NWB + k + LOOK) % NG
                wb = (NWB + k) % NWB
                start_gather(ci + LOOK, gb_next)
                wait_gather(ci, gb)
                wait_write(wb)
                scale(gb, wb)
                start_write(ci, wb)
            return carry

        n_steady = (n_chunks - NWB - LOOK) // NG
        lax.fori_loop(0, n_steady, lambda g, c: steady(NWB + g * NG, c), 0)

        for k in range(LOOK):
            ci = n_chunks - LOOK + k
            wait_gather(ci, ci % NG)
            wait_write(ci % NWB)
            scale(ci % NG, ci % NWB)
            start_write(ci, ci % NWB)
        for b in range(NWB):
            wait_write(b)

    def run(table, x):
        table_p = jnp.pad(table, ((0, 0), (0, _LANES - D)))
        x3 = x.reshape(NW, n_chunks, C)
        return emb_kernel(table_p, x3).reshape(B, D)

    return run


def kernel(x, table):
    Bdim, T = x.shape
    V, D = table.shape
    run = _build(V, D, Bdim * T)
    out = run(table, x.reshape(-1).astype(jnp.int32))
    return out.reshape(Bdim, T, D)


# speedup vs baseline: 1.2315x; 1.2315x over previous
import functools
import math

import jax
import jax.numpy as jnp
from jax import lax
from jax.experimental import pallas as pl
from jax.experimental.pallas import tpu as pltpu
from jax.experimental.pallas import tpu_sc as plsc

D_MODEL = 64
_SCALE = math.sqrt(D_MODEL)
_LANES = 128


@functools.lru_cache(maxsize=None)
def _build(V, D, B):
    info = plsc.get_sparse_core_info()
    NC, NS, L = info.num_cores, info.num_subcores, info.num_lanes
    NW = NC * NS
    assert B % NW == 0
    b_per_w = B // NW
    C = 128
    assert b_per_w % C == 0
    n_chunks = b_per_w // C
    NG = 4
    NWB = 2
    LOOK = 2
    assert n_chunks % NG == 0
    mesh = plsc.VectorSubcoreMesh(core_axis_name="c", subcore_axis_name="s")

    @functools.partial(
        pl.kernel,
        mesh=mesh,
        out_type=jax.ShapeDtypeStruct((B, D), jnp.float32),
        compiler_params=pltpu.CompilerParams(use_tc_tiling_on_sc=True),
        scratch_types=[
            pltpu.VMEM((n_chunks, C), jnp.int32),
            pltpu.VMEM((NG, C, _LANES), jnp.float32),
            pltpu.VMEM((NWB, C, D), jnp.float32),
            [pltpu.SemaphoreType.DMA] * NG,
            [pltpu.SemaphoreType.DMA] * NWB,
        ],
    )
    def emb_kernel(table_hbm, x_hbm, out_hbm, idx_v, gbuf, wbuf, gsems, wsems):
        wid = lax.axis_index("s") * NC + lax.axis_index("c")
        base = wid * b_per_w
        pltpu.sync_copy(x_hbm.at[wid], idx_v)

        def start_gather(ci, b):
            pltpu.async_copy(table_hbm.at[idx_v.at[ci]], gbuf.at[b], gsems[b])

        def wait_gather(ci, b):
            pltpu.make_async_copy(
                table_hbm.at[idx_v.at[ci]], gbuf.at[b], gsems[b]
            ).wait()

        def wait_write(b):
            pltpu.make_async_copy(
                wbuf.at[b], out_hbm.at[pl.ds(base, C)], wsems[b]
            ).wait()

        def start_write(ci, b):
            pltpu.async_copy(
                wbuf.at[b], out_hbm.at[pl.ds(base + ci * C, C)], wsems[b]
            )

        def scale(gb, wb):
            @plsc.parallel_loop(0, C, unroll=8)
            def _scale_body(r):
                for d in range(D // L):
                    sl = pl.ds(d * L, L)
                    wbuf[wb, r, sl] = gbuf[gb, r, sl] * _SCALE

        for ci in range(LOOK):
            start_gather(ci, ci % NG)

        for ci in range(NWB):
            start_gather(ci + LOOK, (ci + LOOK) % NG)
            wait_gather(ci, ci % NG)
            scale(ci % NG, ci % NWB)
            start_write(ci, ci % NWB)

        def steady(ci0, carry):
            for k in range(NG):
                ci = ci0 + k
                gb = (NWB + k) % NG
                gb_next = (NWB + k + LOOK) % NG
                wb = (NWB + k) % NWB
                start_gather(ci + LOOK, gb_next)
                wait_gather(ci, gb)
                wait_write(wb)
                scale(gb, wb)
                start_write(ci, wb)
            return carry

        n_steady = (n_chunks - NWB - LOOK) // NG
        lax.fori_loop(0, n_steady, lambda g, c: steady(NWB + g * NG, c), 0)

        for k in range(LOOK):
            ci = n_chunks - LOOK + k
            wait_gather(ci, ci % NG)
            wait_write(ci % NWB)
            scale(ci % NG, ci % NWB)
            start_write(ci, ci % NWB)
        for b in range(NWB):
            wait_write(b)

    def run(table, x):
        table_p = jnp.pad(table, ((0, 0), (0, _LANES - D)))
        x3 = x.reshape(NW, n_chunks, C)
        return emb_kernel(table_p, x3)

    return run


def kernel(x, table):
    Bdim, T = x.shape
    V, D = table.shape
    run = _build(V, D, Bdim * T)
    out = run(table, x.reshape(-1).astype(jnp.int32))
    return out.reshape(Bdim, T, D)
